# Initial kernel scaffold; baseline (speedup 1.0000x reference)
import functools

import jax
import jax.numpy as jnp
from jax import lax
from jax.experimental import pallas as pl
from jax.experimental.pallas import tpu as pltpu
from jax.experimental.pallas import tpu_sc as plsc

N_NODES = 10000
D = 128
N_EDGES = 320000

NC = 2
NS = 16
NW = NC * NS

CHUNK = 80
EDGES_PER_TILE = N_EDGES // NW
CHUNKS_PER_TILE = EDGES_PER_TILE // CHUNK
ROWS_PER_TILE = N_NODES // NS


def _sc_body(src_hbm, dst_hbm, val_hbm, emb_hbm, p0_hbm, p1_hbm,
             acc, src_v, dst_v, val_v, rows_v, zbuf, sem):
  cid = lax.axis_index("c")
  sid = lax.axis_index("s")
  wid = cid * NS + sid

  zeros16 = jnp.zeros((16,), jnp.float32)

  def zrow(r, carry):
    for g in range(D // 16):
      zbuf[r, pl.ds(g * 16, 16)] = zeros16
    return carry

  lax.fori_loop(0, ROWS_PER_TILE, zrow, None)
  pltpu.sync_copy(zbuf, acc.at[pl.ds(sid * ROWS_PER_TILE, ROWS_PER_TILE)])
  plsc.subcore_barrier()

  def chunk_body(c, carry):
    base = wid * EDGES_PER_TILE + c * CHUNK
    pltpu.sync_copy(src_hbm.at[pl.ds(base, CHUNK)], src_v)
    pltpu.sync_copy(dst_hbm.at[pl.ds(base, CHUNK)], dst_v)
    pltpu.sync_copy(val_hbm.at[pl.ds(base, CHUNK)], val_v)
    pltpu.async_copy(emb_hbm.at[src_v], rows_v, sem).wait()

    def mul_group(g, c2):
      vals16 = val_v[pl.ds(g * 16, 16)]
      for j in range(16):
        w = jnp.take(vals16, jnp.full((16,), j, jnp.int32),
                     mode="promise_in_bounds")
        e = g * 16 + j
        for cg in range(D // 16):
          sl = pl.ds(cg * 16, 16)
          rows_v[e, sl] = rows_v[e, sl] * w
      return c2

    lax.fori_loop(0, CHUNK // 16, mul_group, None)
    pltpu.sync_copy(rows_v, acc.at[dst_v], add=True)
    return carry

  lax.fori_loop(0, CHUNKS_PER_TILE, chunk_body, None)
  plsc.subcore_barrier()

  row0 = sid * ROWS_PER_TILE

  @pl.when(cid == 0)
  def _():
    pltpu.sync_copy(acc.at[pl.ds(row0, ROWS_PER_TILE)],
                    p0_hbm.at[pl.ds(row0, ROWS_PER_TILE)])

  @pl.when(cid == 1)
  def _():
    pltpu.sync_copy(acc.at[pl.ds(row0, ROWS_PER_TILE)],
                    p1_hbm.at[pl.ds(row0, ROWS_PER_TILE)])


_sc_spmm = functools.partial(
    pl.kernel,
    out_type=(jax.ShapeDtypeStruct((N_NODES, D), jnp.float32),
              jax.ShapeDtypeStruct((N_NODES, D), jnp.float32)),
    mesh=plsc.VectorSubcoreMesh(core_axis_name="c", subcore_axis_name="s",
                                num_cores=NC, num_subcores=NS),
    scratch_types=[
        pltpu.VMEM_SHARED((N_NODES, D), jnp.float32),
        pltpu.VMEM((CHUNK,), jnp.int32),
        pltpu.VMEM((CHUNK,), jnp.int32),
        pltpu.VMEM((CHUNK,), jnp.float32),
        pltpu.VMEM((CHUNK, D), jnp.float32),
        pltpu.VMEM((ROWS_PER_TILE, D), jnp.float32),
        pltpu.SemaphoreType.DMA,
    ],
)(_sc_body)


def _add_body(a_ref, b_ref, o_ref):
  o_ref[...] = a_ref[...] + b_ref[...]


def _combine(p0, p1):
  return pl.pallas_call(
      _add_body,
      out_shape=jax.ShapeDtypeStruct((N_NODES, D), jnp.float32),
      grid=(8,),
      in_specs=[pl.BlockSpec((N_NODES // 8, D), lambda i: (i, 0))] * 2,
      out_specs=pl.BlockSpec((N_NODES // 8, D), lambda i: (i, 0)),
  )(p0, p1)


def kernel(edge_index, edge_values, embeds):
  dst = edge_index[0].astype(jnp.int32)
  src = edge_index[1].astype(jnp.int32)
  vals = edge_values.astype(jnp.float32)
  p0, p1 = _sc_spmm(src, dst, vals, embeds)
  return _combine(p0, p1)


# speedup vs baseline: 4.5298x; 4.5298x over previous
import functools

import jax
import jax.numpy as jnp
from jax import lax
from jax.experimental import pallas as pl
from jax.experimental.pallas import tpu as pltpu
from jax.experimental.pallas import tpu_sc as plsc

N_NODES = 10000
D = 128
N_EDGES = 320000

NC = 2
NS = 16
NW = NC * NS

CHUNK = 80
EDGES_PER_TILE = N_EDGES // NW
CHUNKS_PER_TILE = EDGES_PER_TILE // CHUNK
SLAB = 624
REM = N_NODES - SLAB * NS


def _sc_body(src_hbm, dst_hbm, val_hbm, emb_hbm, p0_hbm, p1_hbm,
             acc, src_v, dst_v, val_v, rows_v, sem):
  cid = lax.axis_index("c")
  sid = lax.axis_index("s")
  wid = cid * NS + sid

  zeros16 = jnp.zeros((16,), jnp.float32)

  def zrow(r, carry):
    for g in range(D // 16):
      rows_v[r, pl.ds(g * 16, 16)] = zeros16
    return carry

  lax.fori_loop(0, CHUNK, zrow, None)
  for k in range(SLAB // CHUNK):
    pltpu.sync_copy(rows_v, acc.at[pl.ds(sid * SLAB + k * CHUNK, CHUNK)])
  tail = SLAB % CHUNK
  if tail:
    pltpu.sync_copy(rows_v.at[pl.ds(0, tail)],
                    acc.at[pl.ds(sid * SLAB + SLAB - tail, tail)])

  @pl.when(sid == NS - 1)
  def _():
    pltpu.sync_copy(rows_v.at[pl.ds(0, REM)], acc.at[pl.ds(SLAB * NS, REM)])

  plsc.subcore_barrier()

  def chunk_body(c, carry):
    base = wid * EDGES_PER_TILE + c * CHUNK
    pltpu.sync_copy(src_hbm.at[pl.ds(base, CHUNK)], src_v)
    pltpu.sync_copy(dst_hbm.at[pl.ds(base, CHUNK)], dst_v)
    pltpu.sync_copy(val_hbm.at[pl.ds(base, CHUNK)], val_v)
    pltpu.async_copy(emb_hbm.at[src_v], rows_v, sem).wait()

    def mul_group(g, c2):
      vals16 = val_v[pl.ds(g * 16, 16)]
      dn = lax.GatherDimensionNumbers(offset_dims=(), collapsed_slice_dims=(0,),
                                      start_index_map=(0,))
      for j in range(16):
        w = lax.gather(vals16, jnp.full((16, 1), j, jnp.int32), dn,
                       slice_sizes=(1,),
                       mode=lax.GatherScatterMode.PROMISE_IN_BOUNDS)
        e = g * 16 + j
        for cg in range(D // 16):
          sl = pl.ds(cg * 16, 16)
          rows_v[e, sl] = rows_v[e, sl] * w
      return c2

    lax.fori_loop(0, CHUNK // 16, mul_group, None)
    pltpu.sync_copy(rows_v, acc.at[dst_v], add=True)
    return carry

  lax.fori_loop(0, CHUNKS_PER_TILE, chunk_body, None)
  plsc.subcore_barrier()

  row0 = sid * SLAB

  @pl.when(cid == 0)
  def _():
    pltpu.sync_copy(acc.at[pl.ds(row0, SLAB)], p0_hbm.at[pl.ds(row0, SLAB)])

    @pl.when(sid == NS - 1)
    def _():
      pltpu.sync_copy(acc.at[pl.ds(SLAB * NS, REM)],
                      p0_hbm.at[pl.ds(SLAB * NS, REM)])

  @pl.when(cid == 1)
  def _():
    pltpu.sync_copy(acc.at[pl.ds(row0, SLAB)], p1_hbm.at[pl.ds(row0, SLAB)])

    @pl.when(sid == NS - 1)
    def _():
      pltpu.sync_copy(acc.at[pl.ds(SLAB * NS, REM)],
                      p1_hbm.at[pl.ds(SLAB * NS, REM)])


_sc_spmm = functools.partial(
    pl.kernel,
    out_type=(jax.ShapeDtypeStruct((N_NODES, D), jnp.float32),
              jax.ShapeDtypeStruct((N_NODES, D), jnp.float32)),
    mesh=plsc.VectorSubcoreMesh(core_axis_name="c", subcore_axis_name="s",
                                num_cores=NC, num_subcores=NS),
    scratch_types=[
        pltpu.VMEM_SHARED((N_NODES, D), jnp.float32),
        pltpu.VMEM((CHUNK,), jnp.int32),
        pltpu.VMEM((CHUNK,), jnp.int32),
        pltpu.VMEM((CHUNK,), jnp.float32),
        pltpu.VMEM((CHUNK, D), jnp.float32),
        pltpu.SemaphoreType.DMA,
    ],
)(_sc_body)


def _add_body(a_ref, b_ref, o_ref):
  o_ref[...] = a_ref[...] + b_ref[...]


def _combine(p0, p1):
  return pl.pallas_call(
      _add_body,
      out_shape=jax.ShapeDtypeStruct((N_NODES, D), jnp.float32),
      grid=(10,),
      in_specs=[pl.BlockSpec((N_NODES // 10, D), lambda i: (i, 0))] * 2,
      out_specs=pl.BlockSpec((N_NODES // 10, D), lambda i: (i, 0)),
  )(p0, p1)


def kernel(edge_index, edge_values, embeds):
  dst = edge_index[0].astype(jnp.int32)
  src = edge_index[1].astype(jnp.int32)
  vals = edge_values.astype(jnp.float32)
  p0, p1 = _sc_spmm(src, dst, vals, embeds)
  return _combine(p0, p1)
